# speedup vs baseline: 1.2427x; 1.2427x over previous
import functools

import jax
import jax.numpy as jnp
from jax import lax
from jax.experimental import pallas as pl
from jax.experimental.pallas import tpu as pltpu
from jax.experimental.pallas import tpu_sc as plsc

_D = 768
_NC = 2
_NS = 16
_NW = _NC * _NS
_CHUNK = 64


def _gather_rows(idx3, table):
    nw, nchunks, chunk = idx3.shape
    b_total = nw * nchunks * chunk
    mesh = plsc.VectorSubcoreMesh(core_axis_name="c", subcore_axis_name="s")

    @functools.partial(
        pl.kernel,
        mesh=mesh,
        out_type=jax.ShapeDtypeStruct((b_total, _D), jnp.float32),
        scratch_types=[
            pltpu.VMEM((nchunks, chunk), jnp.int32),
            pltpu.VMEM((chunk, _D), jnp.float32),
            pltpu.SemaphoreType.DMA,
        ],
    )
    def k(idx_hbm, table_hbm, out_hbm, idx_v, rows_v, sem):
        wid = lax.axis_index("s") * _NC + lax.axis_index("c")
        base = wid * nchunks * chunk
        pltpu.sync_copy(idx_hbm.at[wid], idx_v)

        def body(g, carry):
            pltpu.async_copy(table_hbm.at[idx_v.at[g]], rows_v, sem).wait()
            pltpu.sync_copy(rows_v, out_hbm.at[pl.ds(base + g * chunk, chunk)])
            return carry

        lax.fori_loop(0, nchunks, body, 0)

    return k(idx3, table)


def kernel(input_ids, table):
    b, s = input_ids.shape
    idx = input_ids.reshape(-1).astype(jnp.int32)
    nchunks = (b * s) // (_NW * _CHUNK)
    idx3 = idx.reshape(_NW, nchunks, _CHUNK)
    out = _gather_rows(idx3, table)
    return out.reshape(b, s, _D)


# speedup vs baseline: 1.2724x; 1.0239x over previous
import functools

import jax
import jax.numpy as jnp
from jax import lax
from jax.experimental import pallas as pl
from jax.experimental.pallas import tpu as pltpu
from jax.experimental.pallas import tpu_sc as plsc

_D = 768
_NC = 2
_NS = 16
_NW = _NC * _NS
_CHUNK = 64


def _gather_rows(idx3, table):
    nw, nchunks, chunk = idx3.shape
    b_total = nw * nchunks * chunk
    mesh = plsc.VectorSubcoreMesh(core_axis_name="c", subcore_axis_name="s")

    @functools.partial(
        pl.kernel,
        mesh=mesh,
        out_type=jax.ShapeDtypeStruct((b_total, _D), jnp.float32),
        scratch_types=[
            pltpu.VMEM((nchunks, chunk), jnp.int32),
            pltpu.VMEM((chunk, _D), jnp.float32),
            pltpu.VMEM((chunk, _D), jnp.float32),
            pltpu.SemaphoreType.DMA,
            pltpu.SemaphoreType.DMA,
        ],
    )
    def k(idx_hbm, table_hbm, out_hbm, idx_v, rows0_v, rows1_v, gsem, osem):
        wid = lax.axis_index("s") * _NC + lax.axis_index("c")
        base = wid * nchunks * chunk
        pltpu.sync_copy(idx_hbm.at[wid], idx_v)

        bufs = (rows0_v, rows1_v)
        gd = [None] * nchunks
        od = [None] * nchunks
        gd[0] = pltpu.async_copy(table_hbm.at[idx_v.at[0]], bufs[0], gsem)
        for g in range(nchunks):
            buf = bufs[g % 2]
            gd[g].wait()
            od[g] = pltpu.async_copy(
                buf, out_hbm.at[pl.ds(base + g * chunk, chunk)], osem)
            if g + 1 < nchunks:
                if g >= 1:
                    od[g - 1].wait()
                gd[g + 1] = pltpu.async_copy(
                    table_hbm.at[idx_v.at[g + 1]], bufs[(g + 1) % 2], gsem)
        od[nchunks - 2].wait()
        od[nchunks - 1].wait()

    return k(idx3, table)


def kernel(input_ids, table):
    b, s = input_ids.shape
    idx = input_ids.reshape(-1).astype(jnp.int32)
    nchunks = (b * s) // (_NW * _CHUNK)
    idx3 = idx.reshape(_NW, nchunks, _CHUNK)
    out = _gather_rows(idx3, table)
    return out.reshape(b, s, _D)


# speedup vs baseline: 3.3641x; 2.6439x over previous
import functools

import jax
import jax.numpy as jnp
from jax import lax
from jax.experimental import pallas as pl
from jax.experimental.pallas import tpu as pltpu
from jax.experimental.pallas import tpu_sc as plsc

_D = 768
_NC = 2
_NS = 16
_NW = _NC * _NS


def _gather_rows(idx1, table, batch, seq):
    bat_per_w = batch // _NW
    mesh = plsc.VectorSubcoreMesh(core_axis_name="c", subcore_axis_name="s")

    @functools.partial(
        pl.kernel,
        mesh=mesh,
        out_type=jax.ShapeDtypeStruct((seq, batch, _D), jnp.float32),
        scratch_types=[
            pltpu.VMEM((seq * batch,), jnp.int32),
            pltpu.VMEM((bat_per_w, _D), jnp.float32),
            pltpu.VMEM((bat_per_w, _D), jnp.float32),
            pltpu.SemaphoreType.DMA,
            pltpu.SemaphoreType.DMA,
        ],
    )
    def k(idx_hbm, table_hbm, out_hbm, idx_v, rows0_v, rows1_v, gsem, osem):
        wid = lax.axis_index("s") * _NC + lax.axis_index("c")
        base_b = wid * bat_per_w
        pltpu.sync_copy(idx_hbm, idx_v)

        def idx_slice(l):
            return idx_v.at[pl.ds(l * batch + base_b, bat_per_w)]

        bufs = (rows0_v, rows1_v)
        gd = [None] * seq
        od = [None] * seq
        gd[0] = pltpu.async_copy(table_hbm.at[idx_slice(0)], bufs[0], gsem)
        for l in range(seq):
            buf = bufs[l % 2]
            gd[l].wait()
            od[l] = pltpu.async_copy(
                buf, out_hbm.at[l].at[pl.ds(base_b, bat_per_w)], osem)
            if l + 1 < seq:
                if l >= 1:
                    od[l - 1].wait()
                gd[l + 1] = pltpu.async_copy(
                    table_hbm.at[idx_slice(l + 1)], bufs[(l + 1) % 2], gsem)
        od[seq - 2].wait()
        od[seq - 1].wait()

    return k(idx1, table)


def kernel(input_ids, table):
    b, s = input_ids.shape
    idx1 = input_ids.astype(jnp.int32).T.reshape(-1)
    out_t = _gather_rows(idx1, table, b, s)
    return jnp.transpose(out_t, (1, 0, 2))


# speedup vs baseline: 3.6573x; 1.0872x over previous
import functools

import jax
import jax.numpy as jnp
from jax import lax
from jax.experimental import pallas as pl
from jax.experimental.pallas import tpu as pltpu
from jax.experimental.pallas import tpu_sc as plsc

_D = 768
_NC = 2
_NS = 16
_NW = _NC * _NS


def _gather_rows(idx1, table, batch, seq):
    bat_per_w = batch // _NW
    mesh = plsc.VectorSubcoreMesh(core_axis_name="c", subcore_axis_name="s")

    @functools.partial(
        pl.kernel,
        mesh=mesh,
        out_type=jax.ShapeDtypeStruct((seq, batch, _D), jnp.float32),
        scratch_types=[
            pltpu.VMEM((seq * batch,), jnp.int32),
            pltpu.VMEM((bat_per_w, _D), jnp.float32),
            pltpu.VMEM((bat_per_w, _D), jnp.float32),
            pltpu.VMEM((bat_per_w, _D), jnp.float32),
            pltpu.SemaphoreType.DMA,
            pltpu.SemaphoreType.DMA,
        ],
    )
    def k(idx_hbm, table_hbm, out_hbm, idx_v, rows0_v, rows1_v, rows2_v,
          gsem, osem):
        wid = lax.axis_index("s") * _NC + lax.axis_index("c")
        base_b = wid * bat_per_w
        pltpu.sync_copy(idx_hbm, idx_v)

        def idx_slice(l):
            return idx_v.at[pl.ds(l * batch + base_b, bat_per_w)]

        bufs = (rows0_v, rows1_v, rows2_v)
        nb = len(bufs)
        gd = [None] * seq
        od = [None] * seq
        for m in range(nb - 1):
            gd[m] = pltpu.async_copy(
                table_hbm.at[idx_slice(m)], bufs[m % nb], gsem)
        for l in range(seq):
            gd[l].wait()
            od[l] = pltpu.async_copy(
                bufs[l % nb], out_hbm.at[l].at[pl.ds(base_b, bat_per_w)], osem)
            m = l + nb - 1
            if m < seq:
                if l >= 1:
                    od[l - 1].wait()
                gd[m] = pltpu.async_copy(
                    table_hbm.at[idx_slice(m)], bufs[m % nb], gsem)
        for l in range(max(0, seq - nb), seq):
            od[l].wait()

    return k(idx1, table)


def kernel(input_ids, table):
    b, s = input_ids.shape
    idx1 = input_ids.astype(jnp.int32).T.reshape(-1)
    out_t = _gather_rows(idx1, table, b, s)
    return jnp.transpose(out_t, (1, 0, 2))


# speedup vs baseline: 3.7885x; 1.0359x over previous
import functools

import jax
import jax.numpy as jnp
from jax import lax
from jax.experimental import pallas as pl
from jax.experimental.pallas import tpu as pltpu
from jax.experimental.pallas import tpu_sc as plsc

_D = 768
_NC = 2
_NS = 16
_NW = _NC * _NS


def _gather_rows(idx1, table, batch, seq):
    bat_per_w = batch // _NW
    mesh = plsc.VectorSubcoreMesh(core_axis_name="c", subcore_axis_name="s")

    @functools.partial(
        pl.kernel,
        mesh=mesh,
        out_type=jax.ShapeDtypeStruct((seq, batch, _D), jnp.float32),
        scratch_types=[
            pltpu.VMEM((seq * bat_per_w,), jnp.int32),
            pltpu.VMEM((bat_per_w, _D), jnp.float32),
            pltpu.VMEM((bat_per_w, _D), jnp.float32),
            pltpu.VMEM((bat_per_w, _D), jnp.float32),
            pltpu.VMEM((bat_per_w, _D), jnp.float32),
            pltpu.SemaphoreType.DMA,
            pltpu.SemaphoreType.DMA,
        ],
    )
    def k(idx_hbm, table_hbm, out_hbm, idx_v, rows0_v, rows1_v, rows2_v,
          rows3_v, gsem, osem):
        wid = lax.axis_index("s") * _NC + lax.axis_index("c")
        base_b = wid * bat_per_w
        pltpu.sync_copy(
            idx_hbm.at[pl.ds(wid * seq * bat_per_w, seq * bat_per_w)], idx_v)

        def idx_slice(l):
            return idx_v.at[pl.ds(l * bat_per_w, bat_per_w)]

        bufs = (rows0_v, rows1_v, rows2_v, rows3_v)
        nb = len(bufs)
        gd = [None] * seq
        od = [None] * seq
        for m in range(nb - 1):
            gd[m] = pltpu.async_copy(
                table_hbm.at[idx_slice(m)], bufs[m % nb], gsem)
        for l in range(seq):
            gd[l].wait()
            od[l] = pltpu.async_copy(
                bufs[l % nb], out_hbm.at[l].at[pl.ds(base_b, bat_per_w)], osem)
            m = l + nb - 1
            if m < seq:
                if l >= 1:
                    od[l - 1].wait()
                gd[m] = pltpu.async_copy(
                    table_hbm.at[idx_slice(m)], bufs[m % nb], gsem)
        for l in range(max(0, seq - nb), seq):
            od[l].wait()

    return k(idx1, table)


def kernel(input_ids, table):
    b, s = input_ids.shape
    bat_per_w = b // _NW
    idx1 = (input_ids.astype(jnp.int32).T
            .reshape(s, _NW, bat_per_w)
            .transpose(1, 0, 2)
            .reshape(-1))
    out_t = _gather_rows(idx1, table, b, s)
    return jnp.transpose(out_t, (1, 0, 2))
